# baseline (speedup 1.0000x reference)
import jax
import jax.numpy as jnp
from jax.experimental import pallas as pl
from jax.experimental.pallas import tpu as pltpu


def kernel(x):
    B, C, H, W = x.shape
    R = B * C
    CH = 32
    NS = 8
    PT = 8
    NCH = R // CH
    xf = x.reshape(R, H, W)

    def body(x_hbm, o_hbm, buf, fb, lsem, ssem, fsem):
        def cp(src_r0, dst_r0, n, c):
            s = c % NS
            return pltpu.make_async_copy(
                x_hbm.at[pl.ds(src_r0, n)],
                buf.at[pl.ds(CH * s + dst_r0, n)], lsem.at[s])

        def st(loc_r0, out_r0, n, c):
            s = c % NS
            return pltpu.make_async_copy(
                buf.at[pl.ds(CH * s + loc_r0, n)],
                o_hbm.at[pl.ds(out_r0, n)], ssem.at[s])

        def pieces(c):
            r0 = CH * c
            if c in (0, NCH - 1):
                ps = []
                for p in range(CH // PT):
                    off = p * PT
                    if c % 2 == 0 and off == 0:
                        ps.append((
                            [cp(r0, 0, 1, c), cp(r0 + 2, 2, PT - 2, c)],
                            [st(0, r0, 1, c), st(2, r0 + 2, PT - 2, c)]))
                    else:
                        ps.append(([cp(r0 + off, off, PT, c)],
                                   [st(off, r0 + off, PT, c)]))
                return ps
            if c % 2 == 0:
                return [(
                    [cp(r0, 0, 1, c), cp(r0 + 2, 2, CH - 2, c)],
                    [st(0, r0, 1, c), st(2, r0 + 2, CH - 2, c)])]
            return [([cp(r0, 0, CH, c)], [st(0, r0, CH, c)])]

        sched = {c: pieces(c) for c in range(NCH)}

        for c in range(NS):
            for lds, _ in sched[c]:
                for d in lds:
                    d.start()
        fb[...] = jnp.full((1, H, W), 4.0, x.dtype)
        fills = [pltpu.make_async_copy(fb, o_hbm.at[pl.ds(64 * k + 1, 1)], fsem)
                 for k in range(R // 64)]
        for d in fills:
            d.start()

        for c in range(NCH):
            for lds, sts in sched[c]:
                for d in lds:
                    d.wait()
                for d in sts:
                    d.start()
            if c + NS < NCH:
                for _, sts in sched[c]:
                    for d in sts:
                        d.wait()
                for lds, _ in sched[c + NS]:
                    for d in lds:
                        d.start()
        for c in range(NCH - NS, NCH):
            for _, sts in sched[c]:
                for d in sts:
                    d.wait()
        for d in fills:
            d.wait()

    out = pl.pallas_call(
        body,
        in_specs=[pl.BlockSpec(memory_space=pl.ANY)],
        out_specs=pl.BlockSpec(memory_space=pl.ANY),
        out_shape=jax.ShapeDtypeStruct((R, H, W), x.dtype),
        scratch_shapes=[
            pltpu.VMEM((NS * CH, H, W), x.dtype),
            pltpu.VMEM((1, H, W), x.dtype),
            pltpu.SemaphoreType.DMA((NS,)),
            pltpu.SemaphoreType.DMA((NS,)),
            pltpu.SemaphoreType.DMA,
        ],
        compiler_params=pltpu.CompilerParams(
            vmem_limit_bytes=100 * 1024 * 1024),
    )(xf)
    return out.reshape(B, C, H, W)


# speedup vs baseline: 1.0016x; 1.0016x over previous
import jax
import jax.numpy as jnp
from jax.experimental import pallas as pl
from jax.experimental.pallas import tpu as pltpu


def kernel(x):
    B, C, H, W = x.shape
    R = B * C
    CH = 32
    NS = 8
    NCH = R // CH
    xf = x.reshape(R, H, W)

    def body(x_hbm, o_hbm, buf, lsem, ssem):
        for s in range(0, NS, 2):
            buf[pl.ds(CH * s + 1, 1)] = jnp.full((1, H, W), 4.0, x.dtype)

        def loads(c):
            s = c % NS
            b0, r0 = CH * s, CH * c
            if c % 2 == 0:
                return [
                    pltpu.make_async_copy(
                        x_hbm.at[pl.ds(r0, 1)], buf.at[pl.ds(b0, 1)],
                        lsem.at[s]),
                    pltpu.make_async_copy(
                        x_hbm.at[pl.ds(r0 + 2, CH - 2)],
                        buf.at[pl.ds(b0 + 2, CH - 2)], lsem.at[s]),
                ]
            return [pltpu.make_async_copy(
                x_hbm.at[pl.ds(r0, CH)], buf.at[pl.ds(b0, CH)], lsem.at[s])]

        def store(c):
            s = c % NS
            return pltpu.make_async_copy(
                buf.at[pl.ds(CH * s, CH)], o_hbm.at[pl.ds(CH * c, CH)],
                ssem.at[s])

        pending = {}
        for c in range(NS):
            pending[c] = loads(c)
            for d in pending[c]:
                d.start()
        stores = {}
        for c in range(NCH):
            for d in pending.pop(c):
                d.wait()
            stores[c] = store(c)
            stores[c].start()
            if c + NS < NCH:
                stores[c].wait()
                pending[c + NS] = loads(c + NS)
                for d in pending[c + NS]:
                    d.start()
        for c in range(NCH - NS, NCH):
            stores[c].wait()

    out = pl.pallas_call(
        body,
        in_specs=[pl.BlockSpec(memory_space=pl.ANY)],
        out_specs=pl.BlockSpec(memory_space=pl.ANY),
        out_shape=jax.ShapeDtypeStruct((R, H, W), x.dtype),
        scratch_shapes=[
            pltpu.VMEM((NS * CH, H, W), x.dtype),
            pltpu.SemaphoreType.DMA((NS,)),
            pltpu.SemaphoreType.DMA((NS,)),
        ],
        compiler_params=pltpu.CompilerParams(
            vmem_limit_bytes=100 * 1024 * 1024),
    )(xf)
    return out.reshape(B, C, H, W)
